# baseline (speedup 1.0000x reference)
import jax
import jax.numpy as jnp
from jax.experimental import pallas as pl
from jax.experimental.pallas import tpu as pltpu

NSEG = 8
FOLD = 24
SLABS = (0, 3, 21)


def _fam_copies(x, o, b):
    return [
        (x.at[b, 0:7, 8:24], o.at[b, 1:8, 8:24]),
        (x.at[b, 7:8, 8:24], o.at[b, 0:1, 8:24]),
        (x.at[b, :, 32:168], o.at[b, :, 32:168]),
        (x.at[b, 1:8, 176:197], o.at[b, 0:7, 176:197]),
        (x.at[b, 0:1, 176:197], o.at[b, 7:8, 176:197]),
    ]


def _body(x_hbm, o_hbm, in_buf, out_buf, in_sems, out_sems, fam_sems):
    b = pl.program_id(0)
    nb = pl.num_programs(0)
    slot = jax.lax.rem(b, 2)

    def in_copies(step, slot_):
        return [
            pltpu.make_async_copy(
                x_hbm.at[step, :, tb * 8:(tb + 1) * 8, :],
                in_buf.at[slot_, k],
                in_sems.at[slot_],
            )
            for k, tb in enumerate(SLABS)
        ]

    def out_copies(step, slot_):
        return [
            pltpu.make_async_copy(
                out_buf.at[slot_, k],
                o_hbm.at[step, :, tb * 8:(tb + 1) * 8, :],
                out_sems.at[slot_],
            )
            for k, tb in enumerate(SLABS)
        ]

    @pl.when(b == 0)
    def _():
        def fire(i, _):
            for j, (s, d) in enumerate(_fam_copies(x_hbm, o_hbm, i)):
                pltpu.make_async_copy(s, d, fam_sems.at[j]).start()
            return 0

        jax.lax.fori_loop(0, nb, fire, 0)
        for c in in_copies(0, 0):
            c.start()

    @pl.when(b + 1 < nb)
    def _():
        for c in in_copies(b + 1, 1 - slot):
            c.start()

    for c in in_copies(b, slot):
        c.wait()

    @pl.when(b >= 2)
    def _():
        for c in out_copies(b - 2, slot):
            c.wait()

    xin = in_buf[slot]
    fwd = jnp.concatenate([xin[:, NSEG - 1:], xin[:, :NSEG - 1]], axis=1)
    bwd = jnp.concatenate([xin[:, 1:], xin[:, :1]], axis=1)
    r = jax.lax.broadcasted_iota(jnp.int32, xin.shape[1:], dimension=1)
    out0 = jnp.where(r >= 1, fwd[0], xin[0])
    out1 = jnp.where(r == 0, fwd[1], xin[1])
    out2 = jnp.where(r >= 5, bwd[2], xin[2])
    out_buf[slot] = jnp.stack([out0, out1, out2], axis=0)

    for c in out_copies(b, slot):
        c.start()

    @pl.when(b == nb - 1)
    def _():
        @pl.when(b >= 1)
        def _():
            for c in out_copies(b - 1, 1 - slot):
                c.wait()
        for c in out_copies(b, slot):
            c.wait()

        def drain(i, _):
            for j, (s, d) in enumerate(_fam_copies(x_hbm, o_hbm, i)):
                pltpu.make_async_copy(s, d, fam_sems.at[j]).wait()
            return 0

        jax.lax.fori_loop(0, nb, drain, 0)


def kernel(x):
    nt, l, c = x.shape
    nb = nt // NSEG
    xr = x.reshape(nb, NSEG, l, c)
    out = pl.pallas_call(
        _body,
        grid=(nb,),
        in_specs=[pl.BlockSpec(memory_space=pltpu.MemorySpace.HBM)],
        out_specs=pl.BlockSpec(memory_space=pltpu.MemorySpace.HBM),
        out_shape=jax.ShapeDtypeStruct((nb, NSEG, l, c), x.dtype),
        scratch_shapes=[
            pltpu.VMEM((2, 3, NSEG, 8, c), x.dtype),
            pltpu.VMEM((2, 3, NSEG, 8, c), x.dtype),
            pltpu.SemaphoreType.DMA((2,)),
            pltpu.SemaphoreType.DMA((2,)),
            pltpu.SemaphoreType.DMA((5,)),
        ],
    )(xr)
    return out.reshape(nt, l, c)


# speedup vs baseline: 8.1718x; 8.1718x over previous
import functools

import jax
import jax.numpy as jnp
from jax import lax
from jax.experimental import pallas as pl
from jax.experimental.pallas import tpu as pltpu
from jax.experimental.pallas import tpu_sc as plsc

NSEG = 8
FOLD = 24
L = 197
C = 768
NT = 512
NWORK = 32
ROWS_PER_W = NT // NWORK

CHUNKS = (
    (0, 1, 0),
    (1, FOLD, -1),
    (25, 37, 0),
    (62, 37, 0),
    (99, 37, 0),
    (136, 37, 0),
    (173, FOLD, +1),
)
NRING = 3
MAXTOK = 37


def _sc_body(x_hbm, o_hbm, buf, gsem, ssem):
    wid = lax.axis_index("s") * 2 + lax.axis_index("c")
    base = wid * ROWS_PER_W

    transfers = []
    for i in range(ROWS_PER_W):
        t = i % NSEG
        seg_base = (i // NSEG) * NSEG
        for tok0, ntok, shift in CHUNKS:
            src_i = seg_base + (t + shift) % NSEG
            transfers.append((src_i, i, tok0, ntok))

    def gather(q):
        src_i, _, tok0, ntok = transfers[q]
        return pltpu.make_async_copy(
            x_hbm.at[base + src_i, pl.ds(tok0, ntok), :],
            buf.at[q % NRING, 0:ntok, :],
            gsem.at[q % NRING],
        )

    def scatter(q):
        _, dst_i, tok0, ntok = transfers[q]
        return pltpu.make_async_copy(
            buf.at[q % NRING, 0:ntok, :],
            o_hbm.at[base + dst_i, pl.ds(tok0, ntok), :],
            ssem.at[q % NRING],
        )

    n = len(transfers)
    for q in range(n):
        if q >= NRING:
            scatter(q - NRING).wait()
        gather(q).start()
        if q >= 1:
            gather(q - 1).wait()
            scatter(q - 1).start()
    gather(n - 1).wait()
    scatter(n - 1).start()
    for q in range(n - NRING, n):
        scatter(q).wait()


def kernel(x):
    nt, l, c = x.shape
    run = functools.partial(
        pl.kernel,
        out_type=jax.ShapeDtypeStruct((nt, l, c), x.dtype),
        mesh=plsc.VectorSubcoreMesh(core_axis_name="c", subcore_axis_name="s"),
        scratch_types=[
            pltpu.VMEM((NRING, MAXTOK, c), x.dtype),
            pltpu.SemaphoreType.DMA((NRING,)),
            pltpu.SemaphoreType.DMA((NRING,)),
        ],
        compiler_params=pltpu.CompilerParams(use_tc_tiling_on_sc=False),
    )(_sc_body)
    return run(x)


# speedup vs baseline: 12.3088x; 1.5063x over previous
import functools

import jax
import jax.numpy as jnp
from jax import lax
from jax.experimental import pallas as pl
from jax.experimental.pallas import tpu as pltpu
from jax.experimental.pallas import tpu_sc as plsc

NSEG = 8
L = 197
C = 768
NT = 512
NWORK = 32
ROWS_PER_W = NT // NWORK
NBATCH_W = ROWS_PER_W // NSEG

CHUNKS = (
    (8, 16, -1),
    (32, 24, 0),
    (56, 24, 0),
    (80, 24, 0),
    (104, 24, 0),
    (128, 24, 0),
    (152, 16, 0),
    (176, 16, +1),
)
TAIL = (192, 5, +1)
NRING = 3
MAXTOK = 24

SLABS = (
    (0, 1, 7, -1),
    (24, 0, 1, -1),
    (168, 5, 3, +1),
)
NCH = C // 16


def _vcopy_tok(dst_ref, d0, d1, src_ref, s0, s1):
    for k in range(NCH):
        dst_ref[d0, d1, pl.ds(k * 16, 16)] = src_ref[s0, s1, pl.ds(k * 16, 16)]


def _sc_body(x_hbm, o_hbm, buf, tail0, tail1, sbuf, tmp,
             gsem, ssem, tgsem, tssem, slabsem):
    wid = lax.axis_index("s") * 2 + lax.axis_index("c")
    base = wid * ROWS_PER_W

    transfers = []
    main_q = 0
    tail_q = 0
    for i in range(ROWS_PER_W):
        t = i % NSEG
        seg_base = (i // NSEG) * NSEG
        for tok0, ntok, shift in CHUNKS:
            src_i = seg_base + (t + shift) % NSEG
            transfers.append((src_i, i, tok0, ntok, ('main', main_q % NRING)))
            main_q += 1
        tok0, ntok, shift = TAIL
        src_i = seg_base + (t + shift) % NSEG
        transfers.append((src_i, i, tok0, ntok, ('tail', tail_q % 2)))
        tail_q += 1

    def vbuf(ring):
        kind, slot = ring
        if kind == 'main':
            return None, slot
        return (tail0 if slot == 0 else tail1), slot

    def gather(q):
        src_i, _, tok0, ntok, ring = transfers[q]
        tbuf, slot = vbuf(ring)
        if tbuf is None:
            dst = buf.at[slot, 0:ntok, :]
            sem = gsem.at[slot]
        else:
            dst = tbuf
            sem = tgsem.at[slot]
        return pltpu.make_async_copy(
            x_hbm.at[base + src_i, pl.ds(tok0, ntok), :], dst, sem)

    def scatter(q):
        _, dst_i, tok0, ntok, ring = transfers[q]
        tbuf, slot = vbuf(ring)
        if tbuf is None:
            src = buf.at[slot, 0:ntok, :]
            sem = ssem.at[slot]
        else:
            src = tbuf
            sem = tssem.at[slot]
        return pltpu.make_async_copy(
            src, o_hbm.at[base + dst_i, pl.ds(tok0, ntok), :], sem)

    n = len(transfers)
    prev_use = {}
    last_use = {}
    prev_q = [None] * n
    for q, tr in enumerate(transfers):
        key = tr[4]
        prev_q[q] = prev_use.get(key)
        prev_use[key] = q
        last_use[key] = q

    for q in range(n):
        if prev_q[q] is not None:
            scatter(prev_q[q]).wait()
        gather(q).start()
        if q >= 1:
            gather(q - 1).wait()
            scatter(q - 1).start()
    gather(n - 1).wait()
    scatter(n - 1).start()
    for q in sorted(last_use.values()):
        scatter(q).wait()

    for bb in range(NBATCH_W):
        r0 = base + bb * NSEG
        for tok0, m0, nmov, shift in SLABS:
            slab_in = pltpu.make_async_copy(
                x_hbm.at[pl.ds(r0, NSEG), pl.ds(tok0, 8), :], sbuf, slabsem)
            slab_in.start()
            slab_in.wait()
            if shift == -1:
                for j in range(nmov):
                    _vcopy_tok(tmp, 0, j, sbuf, NSEG - 1, m0 + j)

                def body_fwd(i, _, m0=m0, nmov=nmov):
                    t = NSEG - 1 - i
                    for j in range(nmov):
                        _vcopy_tok(sbuf, t, m0 + j, sbuf, t - 1, m0 + j)
                    return 0

                lax.fori_loop(0, NSEG - 1, body_fwd, 0)
                for j in range(nmov):
                    _vcopy_tok(sbuf, 0, m0 + j, tmp, 0, j)
            else:
                for j in range(nmov):
                    _vcopy_tok(tmp, 0, j, sbuf, 0, m0 + j)

                def body_bwd(t, _, m0=m0, nmov=nmov):
                    for j in range(nmov):
                        _vcopy_tok(sbuf, t, m0 + j, sbuf, t + 1, m0 + j)
                    return 0

                lax.fori_loop(0, NSEG - 1, body_bwd, 0)
                for j in range(nmov):
                    _vcopy_tok(sbuf, NSEG - 1, m0 + j, tmp, 0, j)
            slab_out = pltpu.make_async_copy(
                sbuf, o_hbm.at[pl.ds(r0, NSEG), pl.ds(tok0, 8), :], slabsem)
            slab_out.start()
            slab_out.wait()


def kernel(x):
    nt, l, c = x.shape
    run = functools.partial(
        pl.kernel,
        out_type=jax.ShapeDtypeStruct((nt, l, c), x.dtype),
        mesh=plsc.VectorSubcoreMesh(core_axis_name="c", subcore_axis_name="s"),
        scratch_types=[
            pltpu.VMEM((NRING, MAXTOK, C), x.dtype),
            pltpu.VMEM((5, C), x.dtype),
            pltpu.VMEM((5, C), x.dtype),
            pltpu.VMEM((NSEG, 8, C), x.dtype),
            pltpu.VMEM((1, 8, C), x.dtype),
            pltpu.SemaphoreType.DMA((NRING,)),
            pltpu.SemaphoreType.DMA((NRING,)),
            pltpu.SemaphoreType.DMA((2,)),
            pltpu.SemaphoreType.DMA((2,)),
            pltpu.SemaphoreType.DMA,
        ],
    )(_sc_body)
    return run(x)


# speedup vs baseline: 36.6479x; 2.9774x over previous
import functools

import jax
import jax.numpy as jnp
import numpy as np
from jax import lax
from jax.experimental import pallas as pl
from jax.experimental.pallas import tpu as pltpu
from jax.experimental.pallas import tpu_sc as plsc

NSEG = 8
FOLD = 24
L = 197
C = 768
NT = 512
NROWS = L * NT
NWORK = 32
RPW = NROWS // NWORK
NRING = 3
K = 48
PIECES = [K] * (RPW // K) + ([RPW % K] if RPW % K else [])


def _perm() -> np.ndarray:
    j = np.arange(NROWS, dtype=np.int64) // NT
    r = np.arange(NROWS, dtype=np.int64) % NT
    b, t = r // NSEG, r % NSEG
    ts = np.where((j >= 1) & (j < 1 + FOLD), (t - 1) % NSEG,
                  np.where(j >= L - FOLD, (t + 1) % NSEG, t))
    return (j * NT + b * NSEG + ts).astype(np.int32)


def _sc_body(x_hbm, perm_hbm, o_hbm, idx, buf, isem, gsem, ssem):
    wid = lax.axis_index("s") * 2 + lax.axis_index("c")
    base = wid * RPW

    c = pltpu.make_async_copy(perm_hbm.at[pl.ds(base, RPW)], idx, isem)
    c.start()
    c.wait()

    offs = []
    o = 0
    for n in PIECES:
        offs.append(o)
        o += n

    def gather(q):
        o, n = offs[q], PIECES[q]
        return pltpu.make_async_copy(
            x_hbm.at[idx.at[pl.ds(o, n)]],
            buf.at[q % NRING, 0:n, :],
            gsem.at[q % NRING],
        )

    def scatter(q):
        o, n = offs[q], PIECES[q]
        return pltpu.make_async_copy(
            buf.at[q % NRING, 0:n, :],
            o_hbm.at[pl.ds(base + o, n), :],
            ssem.at[q % NRING],
        )

    n = len(PIECES)
    for q in range(n):
        if q >= NRING:
            scatter(q - NRING).wait()
        gather(q).start()
        if q >= 1:
            gather(q - 1).wait()
            scatter(q - 1).start()
    gather(n - 1).wait()
    scatter(n - 1).start()
    for q in range(n - NRING, n):
        scatter(q).wait()


def kernel(x):
    nt, l, c = x.shape
    xt = jnp.transpose(x, (1, 0, 2)).reshape(l * nt, c)
    perm = jnp.asarray(_perm())
    run = functools.partial(
        pl.kernel,
        out_type=jax.ShapeDtypeStruct((l * nt, c), x.dtype),
        mesh=plsc.VectorSubcoreMesh(core_axis_name="c", subcore_axis_name="s"),
        scratch_types=[
            pltpu.VMEM((RPW,), jnp.int32),
            pltpu.VMEM((NRING, K, C), x.dtype),
            pltpu.SemaphoreType.DMA,
            pltpu.SemaphoreType.DMA((NRING,)),
            pltpu.SemaphoreType.DMA((NRING,)),
        ],
    )(_sc_body)
    out2 = run(xt, perm)
    return jnp.transpose(out2.reshape(l, nt, c), (1, 0, 2))


# speedup vs baseline: 36.7075x; 1.0016x over previous
import functools

import jax
import jax.numpy as jnp
import numpy as np
from jax import lax
from jax.experimental import pallas as pl
from jax.experimental.pallas import tpu as pltpu
from jax.experimental.pallas import tpu_sc as plsc

NSEG = 8
FOLD = 24
L = 197
C = 768
NT = 512
NROWS = L * NT
NWORK = 32
RPW = NROWS // NWORK
NRING = 4
K = 32
GA = 2
PIECES = [K] * (RPW // K) + ([RPW % K] if RPW % K else [])


def _perm() -> np.ndarray:
    j = np.arange(NROWS, dtype=np.int64) // NT
    r = np.arange(NROWS, dtype=np.int64) % NT
    b, t = r // NSEG, r % NSEG
    ts = np.where((j >= 1) & (j < 1 + FOLD), (t - 1) % NSEG,
                  np.where(j >= L - FOLD, (t + 1) % NSEG, t))
    return (j * NT + b * NSEG + ts).astype(np.int32)


def _sc_body(x_hbm, perm_hbm, o_hbm, idx, buf, isem, gsem, ssem):
    wid = lax.axis_index("s") * 2 + lax.axis_index("c")
    base = wid * RPW

    c = pltpu.make_async_copy(perm_hbm.at[pl.ds(base, RPW)], idx, isem)
    c.start()
    c.wait()

    offs = []
    o = 0
    for n in PIECES:
        offs.append(o)
        o += n

    def gather(q):
        o, n = offs[q], PIECES[q]
        return pltpu.make_async_copy(
            x_hbm.at[idx.at[pl.ds(o, n)]],
            buf.at[q % NRING, 0:n, :],
            gsem.at[q % NRING],
        )

    def scatter(q):
        o, n = offs[q], PIECES[q]
        return pltpu.make_async_copy(
            buf.at[q % NRING, 0:n, :],
            o_hbm.at[pl.ds(base + o, n), :],
            ssem.at[q % NRING],
        )

    n = len(PIECES)
    for q in range(n):
        if q >= NRING:
            scatter(q - NRING).wait()
        gather(q).start()
        if q >= GA:
            gather(q - GA).wait()
            scatter(q - GA).start()
    for q in range(n - GA, n):
        gather(q).wait()
        scatter(q).start()
    for q in range(n - NRING, n):
        scatter(q).wait()


def kernel(x):
    nt, l, c = x.shape
    xt = jnp.transpose(x, (1, 0, 2)).reshape(l * nt, c)
    perm = jnp.asarray(_perm())
    run = functools.partial(
        pl.kernel,
        out_type=jax.ShapeDtypeStruct((l * nt, c), x.dtype),
        mesh=plsc.VectorSubcoreMesh(core_axis_name="c", subcore_axis_name="s"),
        scratch_types=[
            pltpu.VMEM((RPW,), jnp.int32),
            pltpu.VMEM((NRING, K, C), x.dtype),
            pltpu.SemaphoreType.DMA,
            pltpu.SemaphoreType.DMA((NRING,)),
            pltpu.SemaphoreType.DMA((NRING,)),
        ],
    )(_sc_body)
    out2 = run(xt, perm)
    return jnp.transpose(out2.reshape(l, nt, c), (1, 0, 2))
